# baseline (speedup 1.0000x reference)
import functools

import jax
import jax.numpy as jnp
from jax import lax
from jax.experimental import pallas as pl
from jax.experimental.pallas import tpu as pltpu
from jax.experimental.pallas import tpu_sc as plsc

N = 10000
E = 320000
D = 128
NC = 2
NS = 16
CHUNK = 128
CPT = 160
EPAD = NS * CPT * CHUNK
CPW = CPT // 2
STG = 40
NP = 10112
RPT = NP // NS

_mesh = plsc.VectorSubcoreMesh(core_axis_name="c", subcore_axis_name="s")


@functools.partial(
    pl.kernel,
    out_type=jax.ShapeDtypeStruct((NC, NS, NP), jnp.float32),
    mesh=_mesh,
    scratch_types=[
        pltpu.VMEM((CPW, CHUNK), jnp.int32),
        pltpu.VMEM((NP,), jnp.float32),
    ],
    compiler_params=pltpu.CompilerParams(needs_layout_passes=False),
)
def _deg_kernel(dst_hbm, out_hbm, idx_v, hist_v):
    c = lax.axis_index("c")
    s = lax.axis_index("s")

    def zbody(i, carry):
        hist_v[pl.ds(i * 16, 16)] = jnp.zeros((16,), jnp.float32)
        return carry

    lax.fori_loop(0, NP // 16, zbody, 0)
    ones = jnp.ones((16,), jnp.float32)

    pltpu.sync_copy(dst_hbm.at[s].at[pl.ds(c * CPW, CPW)], idx_v)

    def body(j, carry):
        def inner(k, carry2):
            vals = idx_v[j, pl.ds(k * 16, 16)]
            plsc.addupdate_scatter(hist_v, [vals], ones)
            return carry2

        lax.fori_loop(0, CHUNK // 16, inner, 0)
        return carry

    lax.fori_loop(0, CPW, body, 0)
    pltpu.sync_copy(hist_v, out_hbm.at[c].at[s])


@functools.partial(
    pl.kernel,
    out_type=jax.ShapeDtypeStruct((NC, NP, D), jnp.float32),
    mesh=_mesh,
    scratch_types=[
        pltpu.VMEM((STG, CHUNK), jnp.int32),
        pltpu.VMEM((STG, CHUNK), jnp.int32),
        pltpu.VMEM((2, CHUNK, D), jnp.float32),
        pltpu.VMEM_SHARED((NP, D), jnp.float32),
        pltpu.SemaphoreType.DMA,
        pltpu.SemaphoreType.DMA,
    ],
)
def _agg_kernel(g_hbm, src_hbm, dst_hbm, zeros_hbm, out_hbm,
                src_v, dst_v, rows_v, acc_sh, sem0, sem1):
    c = lax.axis_index("c")
    s = lax.axis_index("s")
    pltpu.sync_copy(zeros_hbm, acc_sh.at[pl.ds(s * RPT, RPT)])
    plsc.subcore_barrier()

    def stage(base_row):
        pltpu.sync_copy(src_hbm.at[s].at[pl.ds(base_row, STG)], src_v)
        pltpu.sync_copy(dst_hbm.at[s].at[pl.ds(base_row, STG)], dst_v)
        pltpu.async_copy(g_hbm.at[src_v.at[0]], rows_v.at[0], sem0)
        pltpu.async_copy(g_hbm.at[src_v.at[1]], rows_v.at[1], sem1)

        def body(i, carry):
            j0 = 2 * i
            pltpu.make_async_copy(g_hbm.at[src_v.at[j0]],
                                  rows_v.at[0], sem0).wait()
            pltpu.sync_copy(rows_v.at[0], acc_sh.at[dst_v.at[j0]], add=True)
            pltpu.async_copy(g_hbm.at[src_v.at[j0 + 2]], rows_v.at[0], sem0)
            pltpu.make_async_copy(g_hbm.at[src_v.at[j0 + 1]],
                                  rows_v.at[1], sem1).wait()
            pltpu.sync_copy(rows_v.at[1], acc_sh.at[dst_v.at[j0 + 1]], add=True)
            pltpu.async_copy(g_hbm.at[src_v.at[j0 + 3]], rows_v.at[1], sem1)
            return carry

        lax.fori_loop(0, STG // 2 - 1, body, 0)
        j0 = STG - 2
        pltpu.make_async_copy(g_hbm.at[src_v.at[j0]],
                              rows_v.at[0], sem0).wait()
        pltpu.sync_copy(rows_v.at[0], acc_sh.at[dst_v.at[j0]], add=True)
        pltpu.make_async_copy(g_hbm.at[src_v.at[j0 + 1]],
                              rows_v.at[1], sem1).wait()
        pltpu.sync_copy(rows_v.at[1], acc_sh.at[dst_v.at[j0 + 1]], add=True)

    for t in range(CPW // STG):
        stage(c * CPW + t * STG)

    plsc.subcore_barrier()
    pltpu.sync_copy(acc_sh.at[pl.ds(s * RPT, RPT)],
                    out_hbm.at[c].at[pl.ds(s * RPT, RPT)])


def _tc1_body(x_ref, w1_ref, dct_ref, g_out, dinv_out):
    degsum = lax.dot_general(dct_ref[...], jnp.ones((NC * NS, 1), jnp.float32),
                             (((1,), (0,)), ((), ())),
                             preferred_element_type=jnp.float32)
    deg = degsum[: N] + 1.0
    dinv = lax.rsqrt(deg)
    h = lax.dot_general(x_ref[...], w1_ref[...], (((1,), (1,)), ((), ())),
                        preferred_element_type=jnp.float32)
    g_out[...] = dinv * h
    dinv_out[...] = dinv


def _tc_mid_body(agg_ref, gprev_ref, dinv_ref, b_ref, gam_ref, bet_ref, a_ref,
                 w2_ref, g_out):
    dinv = dinv_ref[...]
    full = agg_ref[0, : N, :] + agg_ref[1, : N, :] + gprev_ref[...]
    conv = dinv * full + b_ref[...]
    m = jnp.mean(conv, axis=0, keepdims=True)
    v = jnp.mean((conv - m) ** 2, axis=0, keepdims=True)
    bn = gam_ref[...] * (conv - m) * lax.rsqrt(v + 1e-5) + bet_ref[...]
    p = jnp.where(bn >= 0, bn, a_ref[0, 0] * bn)
    h2 = lax.dot_general(p, w2_ref[...], (((1,), (1,)), ((), ())),
                         preferred_element_type=jnp.float32)
    g_out[...] = dinv * h2


def _tc_final_body(agg_ref, gprev_ref, dinv_ref, b_ref, gam_ref, bet_ref,
                   a_ref, out_ref):
    dinv = dinv_ref[...]
    full = agg_ref[0, : N, :] + agg_ref[1, : N, :] + gprev_ref[...]
    conv = dinv * full + b_ref[...]
    m = jnp.mean(conv, axis=0, keepdims=True)
    v = jnp.mean((conv - m) ** 2, axis=0, keepdims=True)
    bn = gam_ref[...] * (conv - m) * lax.rsqrt(v + 1e-5) + bet_ref[...]
    out_ref[...] = jnp.where(bn >= 0, bn, a_ref[0, 0] * bn)


_tc1 = pl.pallas_call(
    _tc1_body,
    out_shape=[jax.ShapeDtypeStruct((N, D), jnp.float32),
               jax.ShapeDtypeStruct((N, 1), jnp.float32)],
)

_tc_mid = pl.pallas_call(
    _tc_mid_body,
    out_shape=jax.ShapeDtypeStruct((N, D), jnp.float32),
)

_tc_final = pl.pallas_call(
    _tc_final_body,
    out_shape=jax.ShapeDtypeStruct((N, D), jnp.float32),
)


def kernel(x, edge_index, W1, b1, g1, be1, a1, W2, b2, g2, be2, a2):
    src = edge_index[0]
    dst = edge_index[1]
    pad = EPAD - E
    srcp = jnp.concatenate([src, jnp.zeros((pad,), jnp.int32)])
    dstp = jnp.concatenate(
        [dst, N + (jnp.arange(pad, dtype=jnp.int32) % (NP - N))])
    src_rs = srcp.reshape(NS, CPT, CHUNK)
    dst_rs = dstp.reshape(NS, CPT, CHUNK)

    zerosD = jnp.zeros((RPT, D), jnp.float32)

    dc = _deg_kernel(dst_rs)
    dct = dc.reshape(NC * NS, NP).T
    g1s, dinv = _tc1(x, W1, dct)
    agg1 = _agg_kernel(g1s, src_rs, dst_rs, zerosD)
    g2s = _tc_mid(agg1, g1s, dinv, b1.reshape(1, D), g1.reshape(1, D),
                  be1.reshape(1, D), a1.reshape(1, 1), W2)
    agg2 = _agg_kernel(g2s, src_rs, dst_rs, zerosD)
    return _tc_final(agg2, g2s, dinv, b2.reshape(1, D), g2.reshape(1, D),
                     be2.reshape(1, D), a2.reshape(1, 1))


# speedup vs baseline: 1.1803x; 1.1803x over previous
import functools

import jax
import jax.numpy as jnp
from jax import lax
from jax.experimental import pallas as pl
from jax.experimental.pallas import tpu as pltpu
from jax.experimental.pallas import tpu_sc as plsc

N = 10000
E = 320000
D = 128
NC = 2
NS = 16
CHUNK = 128
CPT = 160
EPAD = NS * CPT * CHUNK
CPW = CPT // 2
STG = 32
AGG0 = 4
AGG1 = 1
NP = 10112
RPT = NP // NS

_mesh = plsc.VectorSubcoreMesh(core_axis_name="c", subcore_axis_name="s")


@functools.partial(
    pl.kernel,
    out_type=jax.ShapeDtypeStruct((NC, NS, NP), jnp.float32),
    mesh=_mesh,
    scratch_types=[
        pltpu.VMEM((CPW, CHUNK), jnp.int32),
        pltpu.VMEM((NP,), jnp.float32),
    ],
    compiler_params=pltpu.CompilerParams(needs_layout_passes=False),
)
def _deg_kernel(dst_hbm, out_hbm, idx_v, hist_v):
    c = lax.axis_index("c")
    s = lax.axis_index("s")

    def zbody(i, carry):
        hist_v[pl.ds(i * 16, 16)] = jnp.zeros((16,), jnp.float32)
        return carry

    lax.fori_loop(0, NP // 16, zbody, 0)
    ones = jnp.ones((16,), jnp.float32)

    pltpu.sync_copy(dst_hbm.at[s].at[pl.ds(c * CPW, CPW)], idx_v)

    def body(j, carry):
        def inner(k, carry2):
            vals = idx_v[j, pl.ds(k * 16, 16)]
            plsc.addupdate_scatter(hist_v, [vals], ones)
            return carry2

        lax.fori_loop(0, CHUNK // 16, inner, 0)
        return carry

    lax.fori_loop(0, CPW, body, 0)
    pltpu.sync_copy(hist_v, out_hbm.at[c].at[s])


@functools.partial(
    pl.kernel,
    out_type=jax.ShapeDtypeStruct((NC, NP, D), jnp.float32),
    mesh=_mesh,
    scratch_types=[
        pltpu.VMEM((STG, CHUNK), jnp.int32),
        pltpu.VMEM((STG, CHUNK), jnp.int32),
        pltpu.VMEM((2, CHUNK, D), jnp.float32),
        pltpu.VMEM_SHARED((NP, D), jnp.float32),
        pltpu.SemaphoreType.DMA,
        pltpu.SemaphoreType.DMA,
    ],
)
def _agg_kernel(g_hbm, src_hbm, dst_hbm, zeros_hbm, out_hbm,
                src_v, dst_v, rows_v, acc_sh, sem0, sem1):
    c = lax.axis_index("c")
    s = lax.axis_index("s")
    pltpu.sync_copy(zeros_hbm, acc_sh.at[pl.ds(s * RPT, RPT)])
    plsc.subcore_barrier()

    def stage(base_row):
        pltpu.sync_copy(src_hbm.at[s].at[pl.ds(base_row, STG)], src_v)
        pltpu.sync_copy(dst_hbm.at[s].at[pl.ds(base_row, STG)], dst_v)
        pltpu.async_copy(g_hbm.at[src_v.at[0]], rows_v.at[0], sem0)
        pltpu.async_copy(g_hbm.at[src_v.at[1]], rows_v.at[1], sem1)

        def body(i, carry):
            j0 = 2 * i
            pltpu.make_async_copy(g_hbm.at[src_v.at[j0]],
                                  rows_v.at[0], sem0).wait()
            pltpu.sync_copy(rows_v.at[0], acc_sh.at[dst_v.at[j0]], add=True)
            pltpu.async_copy(g_hbm.at[src_v.at[j0 + 2]], rows_v.at[0], sem0)
            pltpu.make_async_copy(g_hbm.at[src_v.at[j0 + 1]],
                                  rows_v.at[1], sem1).wait()
            pltpu.sync_copy(rows_v.at[1], acc_sh.at[dst_v.at[j0 + 1]], add=True)
            pltpu.async_copy(g_hbm.at[src_v.at[j0 + 3]], rows_v.at[1], sem1)
            return carry

        lax.fori_loop(0, STG // 2 - 1, body, 0)
        j0 = STG - 2
        pltpu.make_async_copy(g_hbm.at[src_v.at[j0]],
                              rows_v.at[0], sem0).wait()
        pltpu.sync_copy(rows_v.at[0], acc_sh.at[dst_v.at[j0]], add=True)
        pltpu.make_async_copy(g_hbm.at[src_v.at[j0 + 1]],
                              rows_v.at[1], sem1).wait()
        pltpu.sync_copy(rows_v.at[1], acc_sh.at[dst_v.at[j0 + 1]], add=True)

    @pl.when(c == 0)
    def _():
        for t in range(AGG0):
            stage(t * STG)

    @pl.when(c == 1)
    def _():
        for t in range(AGG1):
            stage((AGG0 + t) * STG)

    plsc.subcore_barrier()
    pltpu.sync_copy(acc_sh.at[pl.ds(s * RPT, RPT)],
                    out_hbm.at[c].at[pl.ds(s * RPT, RPT)])


def _tc1_body(x_ref, w1_ref, dct_ref, g_out, dinv_out):
    degsum = lax.dot_general(dct_ref[...], jnp.ones((NC * NS, 1), jnp.float32),
                             (((1,), (0,)), ((), ())),
                             preferred_element_type=jnp.float32)
    deg = degsum[: N] + 1.0
    dinv = lax.rsqrt(deg)
    h = lax.dot_general(x_ref[...], w1_ref[...], (((1,), (1,)), ((), ())),
                        preferred_element_type=jnp.float32)
    g_out[...] = dinv * h
    dinv_out[...] = dinv


def _tc_mid_body(agg_ref, gprev_ref, dinv_ref, b_ref, gam_ref, bet_ref, a_ref,
                 w2_ref, g_out):
    dinv = dinv_ref[...]
    full = agg_ref[0, : N, :] + agg_ref[1, : N, :] + gprev_ref[...]
    conv = dinv * full + b_ref[...]
    m = jnp.mean(conv, axis=0, keepdims=True)
    v = jnp.mean((conv - m) ** 2, axis=0, keepdims=True)
    bn = gam_ref[...] * (conv - m) * lax.rsqrt(v + 1e-5) + bet_ref[...]
    p = jnp.where(bn >= 0, bn, a_ref[0, 0] * bn)
    h2 = lax.dot_general(p, w2_ref[...], (((1,), (1,)), ((), ())),
                         preferred_element_type=jnp.float32)
    g_out[...] = dinv * h2


def _tc_final_body(agg_ref, gprev_ref, dinv_ref, b_ref, gam_ref, bet_ref,
                   a_ref, out_ref):
    dinv = dinv_ref[...]
    full = agg_ref[0, : N, :] + agg_ref[1, : N, :] + gprev_ref[...]
    conv = dinv * full + b_ref[...]
    m = jnp.mean(conv, axis=0, keepdims=True)
    v = jnp.mean((conv - m) ** 2, axis=0, keepdims=True)
    bn = gam_ref[...] * (conv - m) * lax.rsqrt(v + 1e-5) + bet_ref[...]
    out_ref[...] = jnp.where(bn >= 0, bn, a_ref[0, 0] * bn)


_tc1 = pl.pallas_call(
    _tc1_body,
    out_shape=[jax.ShapeDtypeStruct((N, D), jnp.float32),
               jax.ShapeDtypeStruct((N, 1), jnp.float32)],
)

_tc_mid = pl.pallas_call(
    _tc_mid_body,
    out_shape=jax.ShapeDtypeStruct((N, D), jnp.float32),
)

_tc_final = pl.pallas_call(
    _tc_final_body,
    out_shape=jax.ShapeDtypeStruct((N, D), jnp.float32),
)


def kernel(x, edge_index, W1, b1, g1, be1, a1, W2, b2, g2, be2, a2):
    src = edge_index[0]
    dst = edge_index[1]
    pad = EPAD - E
    srcp = jnp.concatenate([src, jnp.zeros((pad,), jnp.int32)])
    dstp = jnp.concatenate(
        [dst, N + (jnp.arange(pad, dtype=jnp.int32) % (NP - N))])
    src_rs = srcp.reshape(NS, CPT, CHUNK)
    dst_rs = dstp.reshape(NS, CPT, CHUNK)

    zerosD = jnp.zeros((RPT, D), jnp.float32)

    dc = _deg_kernel(dst_rs)
    dct = dc.reshape(NC * NS, NP).T
    g1s, dinv = _tc1(x, W1, dct)
    agg1 = _agg_kernel(g1s, src_rs, dst_rs, zerosD)
    g2s = _tc_mid(agg1, g1s, dinv, b1.reshape(1, D), g1.reshape(1, D),
                  be1.reshape(1, D), a1.reshape(1, 1), W2)
    agg2 = _agg_kernel(g2s, src_rs, dst_rs, zerosD)
    return _tc_final(agg2, g2s, dinv, b2.reshape(1, D), g2.reshape(1, D),
                     be2.reshape(1, D), a2.reshape(1, 1))
